# baseline (speedup 1.0000x reference)
import functools

import jax
import jax.numpy as jnp
from jax import lax
from jax.experimental import pallas as pl
from jax.experimental.pallas import tpu as pltpu
from jax.experimental.pallas import tpu_sc as plsc

VOCAB = 100000
EDIM = 32
NTAB = 26
B = 16384
NUM_DENSE = 13
CAT_DIM = NTAB * EDIM

BC = 4096
NCHUNK = B // BC
BN = 2048

NT_A = 13
NT_B = NTAB - NT_A
D_A = NT_A * EDIM
D_B = NT_B * EDIM


def _sc_gather_t(tabT, idx, t_base, npart):
    mesh = plsc.VectorSubcoreMesh(core_axis_name="core", subcore_axis_name="subcore")

    @functools.partial(
        pl.kernel,
        out_type=jax.ShapeDtypeStruct((npart * EDIM, B), jnp.float32),
        mesh=mesh,
        compiler_params=pltpu.CompilerParams(needs_layout_passes=False),
        scratch_types=[
            pltpu.VMEM((VOCAB,), jnp.float32),
            pltpu.VMEM((BC,), jnp.int32),
            pltpu.VMEM((BC,), jnp.int32),
            pltpu.VMEM((BC,), jnp.float32),
            pltpu.VMEM((BC,), jnp.float32),
            pltpu.SemaphoreType.DMA,
            pltpu.SemaphoreType.DMA,
            pltpu.SemaphoreType.DMA,
            pltpu.SemaphoreType.DMA,
        ],
    )
    def gather_kernel(tabT_hbm, idx_hbm, out_hbm, tab_v, idx_va, idx_vb,
                      out_va, out_vb, sem_i0, sem_i1, sem_o0, sem_o1):
        e = lax.axis_index("subcore") * 2 + lax.axis_index("core")
        t0 = (e * npart) // 32
        sem_i = (sem_i0, sem_i1)
        sem_o = (sem_o0, sem_o1)
        idx_bufs = (idx_va, idx_vb)
        out_bufs = (out_va, out_vb)

        def start_idx(t, c, buf):
            pltpu.async_copy(
                idx_hbm.at[t, pl.ds(c * BC, BC)], idx_bufs[buf], sem_i[buf])

        def wait_idx(buf):
            pltpu.make_async_copy(
                idx_hbm.at[0, pl.ds(0, BC)], idx_bufs[buf], sem_i[buf]).wait()

        def wait_out(buf):
            pltpu.make_async_copy(
                out_bufs[buf], out_hbm.at[0, pl.ds(0, BC)], sem_o[buf]).wait()

        start_idx(t_base + t0, 0, 0)

        @pl.loop(0, npart)
        def _(k):
            tl = lax.rem(t0 + k, npart)
            t = t_base + tl
            t_next = t_base + lax.rem(t0 + k + 1, npart)
            pltpu.sync_copy(tabT_hbm.at[t, e], tab_v)

            for c in range(NCHUNK):
                buf = c % 2
                wait_idx(buf)
                if c < NCHUNK - 1:
                    start_idx(t, c + 1, (c + 1) % 2)
                else:
                    @pl.when(k < npart - 1)
                    def _():
                        start_idx(t_next, 0, 0)
                if c >= 2:
                    wait_out(buf)
                else:
                    @pl.when(k > 0)
                    def _():
                        wait_out(buf)

                ib, ob = idx_bufs[buf], out_bufs[buf]

                @plsc.parallel_loop(0, BC // 16, unroll=8)
                def _(i):
                    iv = ib[pl.ds(i * 16, 16)]
                    ob[pl.ds(i * 16, 16)] = plsc.load_gather(tab_v, [iv])

                pltpu.async_copy(
                    ob, out_hbm.at[tl * EDIM + e, pl.ds(c * BC, BC)], sem_o[buf])

        wait_out(0)
        wait_out(1)

    return gather_kernel(tabT, idx)


def _full(shape):
    return pl.BlockSpec(shape, lambda i: (0, 0))


def _mlp_s1_body(xTa_ref, numT_ref, w1aT_ref, w1nT_ref, b1_ref, h_ref):
    h = jnp.dot(w1aT_ref[...], xTa_ref[...], preferred_element_type=jnp.float32)
    h = h + jnp.dot(w1nT_ref[...], numT_ref[...], preferred_element_type=jnp.float32)
    h_ref[...] = h + b1_ref[...]


def _mlp_s1(xTa, numT, w1aT, w1nT, b1c):
    return pl.pallas_call(
        _mlp_s1_body,
        grid=(B // BN,),
        in_specs=[
            pl.BlockSpec((D_A, BN), lambda i: (0, i)),
            pl.BlockSpec((NUM_DENSE, BN), lambda i: (0, i)),
            _full((512, D_A)),
            _full((512, NUM_DENSE)),
            _full((512, 1)),
        ],
        out_specs=pl.BlockSpec((512, BN), lambda i: (0, i)),
        out_shape=jax.ShapeDtypeStruct((512, B), jnp.float32),
    )(xTa, numT, w1aT, w1nT, b1c)


def _mlp_s2_body(h1p_ref, xTb_ref, w1bT_ref, w2T_ref, b2_ref, w3T_ref, b3_ref,
                 w4T_ref, b4_ref, outT_ref):
    h = h1p_ref[...] + jnp.dot(
        w1bT_ref[...], xTb_ref[...], preferred_element_type=jnp.float32)
    h = jnp.maximum(h, 0.0)
    h = jnp.maximum(
        jnp.dot(w2T_ref[...], h, preferred_element_type=jnp.float32) + b2_ref[...], 0.0)
    h = jnp.maximum(
        jnp.dot(w3T_ref[...], h, preferred_element_type=jnp.float32) + b3_ref[...], 0.0)
    outT_ref[...] = jnp.dot(w4T_ref[...], h, preferred_element_type=jnp.float32) + b4_ref[...]


def _mlp_s2(h1p, xTb, w1bT, w2T, b2c, w3T, b3c, w4T, b4c):
    return pl.pallas_call(
        _mlp_s2_body,
        grid=(B // BN,),
        in_specs=[
            pl.BlockSpec((512, BN), lambda i: (0, i)),
            pl.BlockSpec((D_B, BN), lambda i: (0, i)),
            _full((512, D_B)),
            _full((256, 512)),
            _full((256, 1)),
            _full((128, 256)),
            _full((128, 1)),
            _full((1, 128)),
            _full((1, 1)),
        ],
        out_specs=pl.BlockSpec((1, BN), lambda i: (0, i)),
        out_shape=jax.ShapeDtypeStruct((1, B), jnp.float32),
    )(h1p, xTb, w1bT, w2T, b2c, w3T, b3c, w4T, b4c)


def kernel(numerical_features, cat_features, tables, W1, b1, W2, b2, W3, b3, W4, b4):
    idx = jnp.mod(cat_features[:, :, 0], VOCAB)
    tabT = jnp.transpose(tables, (0, 2, 1))
    numT = numerical_features.T
    xT_a = _sc_gather_t(tabT, idx, 0, NT_A)
    xT_b = _sc_gather_t(tabT, idx, NT_A, NT_B)
    h1p = _mlp_s1(xT_a, numT, W1[:D_A].T, W1[CAT_DIM:].T, b1.reshape(-1, 1))
    outT = _mlp_s2(
        h1p,
        xT_b,
        W1[D_A:CAT_DIM].T,
        W2.T,
        b2.reshape(-1, 1),
        W3.T,
        b3.reshape(-1, 1),
        W4.T,
        b4.reshape(-1, 1),
    )
    return outT.reshape(B, 1)


# speedup vs baseline: 1.0545x; 1.0545x over previous
import functools

import jax
import jax.numpy as jnp
from jax import lax
from jax.experimental import pallas as pl
from jax.experimental.pallas import tpu as pltpu
from jax.experimental.pallas import tpu_sc as plsc

VOCAB = 100000
EDIM = 32
NTAB = 26
B = 16384
NUM_DENSE = 13
CAT_DIM = NTAB * EDIM

BC = 4096
NCHUNK = B // BC
BN = 2048


def _sc_gather_t(tabT, idx):
    mesh = plsc.VectorSubcoreMesh(core_axis_name="core", subcore_axis_name="subcore")

    @functools.partial(
        pl.kernel,
        out_type=jax.ShapeDtypeStruct((CAT_DIM, B), jnp.float32),
        mesh=mesh,
        compiler_params=pltpu.CompilerParams(needs_layout_passes=False),
        scratch_types=[
            pltpu.VMEM((VOCAB,), jnp.float32),
            pltpu.VMEM((BC,), jnp.int32),
            pltpu.VMEM((BC,), jnp.int32),
            pltpu.VMEM((BC,), jnp.float32),
            pltpu.VMEM((BC,), jnp.float32),
            pltpu.SemaphoreType.DMA,
            pltpu.SemaphoreType.DMA,
            pltpu.SemaphoreType.DMA,
            pltpu.SemaphoreType.DMA,
        ],
    )
    def gather_kernel(tabT_hbm, idx_hbm, out_hbm, tab_v, idx_va, idx_vb,
                      out_va, out_vb, sem_i0, sem_i1, sem_o0, sem_o1):
        e = lax.axis_index("subcore") * 2 + lax.axis_index("core")
        t0 = (e * NTAB) // 32
        sem_i = (sem_i0, sem_i1)
        sem_o = (sem_o0, sem_o1)
        idx_bufs = (idx_va, idx_vb)
        out_bufs = (out_va, out_vb)

        def start_idx(t, c, buf):
            pltpu.async_copy(
                idx_hbm.at[t, pl.ds(c * BC, BC)], idx_bufs[buf], sem_i[buf])

        def wait_idx(buf):
            pltpu.make_async_copy(
                idx_hbm.at[0, pl.ds(0, BC)], idx_bufs[buf], sem_i[buf]).wait()

        def wait_out(buf):
            pltpu.make_async_copy(
                out_bufs[buf], out_hbm.at[0, pl.ds(0, BC)], sem_o[buf]).wait()

        start_idx(t0, 0, 0)

        @pl.loop(0, NTAB)
        def _(k):
            t = lax.rem(t0 + k, NTAB)
            t_next = lax.rem(t0 + k + 1, NTAB)
            pltpu.sync_copy(tabT_hbm.at[t, e], tab_v)

            for c in range(NCHUNK):
                buf = c % 2
                wait_idx(buf)
                if c < NCHUNK - 1:
                    start_idx(t, c + 1, (c + 1) % 2)
                else:
                    @pl.when(k < NTAB - 1)
                    def _():
                        start_idx(t_next, 0, 0)
                if c >= 2:
                    wait_out(buf)
                else:
                    @pl.when(k > 0)
                    def _():
                        wait_out(buf)

                ib, ob = idx_bufs[buf], out_bufs[buf]

                @plsc.parallel_loop(0, BC // 16, unroll=16)
                def _(i):
                    iv = ib[pl.ds(i * 16, 16)]
                    ob[pl.ds(i * 16, 16)] = plsc.load_gather(tab_v, [iv])

                pltpu.async_copy(
                    ob, out_hbm.at[t * EDIM + e, pl.ds(c * BC, BC)], sem_o[buf])

        wait_out(0)
        wait_out(1)

    return gather_kernel(tabT, idx)


def _mlp_t_body(xT_ref, numT_ref, w1cT_ref, w1nT_ref, b1_ref, w2T_ref, b2_ref,
                w3T_ref, b3_ref, w4T_ref, b4_ref, outT_ref):
    h = jnp.dot(w1cT_ref[...], xT_ref[...], preferred_element_type=jnp.float32)
    h = h + jnp.dot(w1nT_ref[...], numT_ref[...], preferred_element_type=jnp.float32)
    h = jnp.maximum(h + b1_ref[...], 0.0)
    h = jnp.maximum(
        jnp.dot(w2T_ref[...], h, preferred_element_type=jnp.float32) + b2_ref[...], 0.0)
    h = jnp.maximum(
        jnp.dot(w3T_ref[...], h, preferred_element_type=jnp.float32) + b3_ref[...], 0.0)
    outT_ref[...] = jnp.dot(w4T_ref[...], h, preferred_element_type=jnp.float32) + b4_ref[...]


def _mlp_t(xT, numT, w1cT, w1nT, b1c, w2T, b2c, w3T, b3c, w4T, b4c):
    nblk = B // BN
    full = lambda shape: pl.BlockSpec(shape, lambda i: (0, 0))
    return pl.pallas_call(
        _mlp_t_body,
        grid=(nblk,),
        in_specs=[
            pl.BlockSpec((CAT_DIM, BN), lambda i: (0, i)),
            pl.BlockSpec((NUM_DENSE, BN), lambda i: (0, i)),
            full((512, CAT_DIM)),
            full((512, NUM_DENSE)),
            full((512, 1)),
            full((256, 512)),
            full((256, 1)),
            full((128, 256)),
            full((128, 1)),
            full((1, 128)),
            full((1, 1)),
        ],
        out_specs=pl.BlockSpec((1, BN), lambda i: (0, i)),
        out_shape=jax.ShapeDtypeStruct((1, B), jnp.float32),
    )(xT, numT, w1cT, w1nT, b1c, w2T, b2c, w3T, b3c, w4T, b4c)


def kernel(numerical_features, cat_features, tables, W1, b1, W2, b2, W3, b3, W4, b4):
    idx = jnp.mod(cat_features[:, :, 0], VOCAB)
    tabT = jnp.transpose(tables, (0, 2, 1))
    xT = _sc_gather_t(tabT, idx)
    numT = numerical_features.T
    outT = _mlp_t(
        xT,
        numT,
        W1[:CAT_DIM].T,
        W1[CAT_DIM:].T,
        b1.reshape(-1, 1),
        W2.T,
        b2.reshape(-1, 1),
        W3.T,
        b3.reshape(-1, 1),
        W4.T,
        b4.reshape(-1, 1),
    )
    return outT.reshape(B, 1)


# speedup vs baseline: 1.1701x; 1.1097x over previous
import functools

import jax
import jax.numpy as jnp
from jax import lax
from jax.experimental import pallas as pl
from jax.experimental.pallas import tpu as pltpu
from jax.experimental.pallas import tpu_sc as plsc

VOCAB = 100000
EDIM = 32
NTAB = 26
B = 16384
NUM_DENSE = 13
CAT_DIM = NTAB * EDIM

BC = 4096
NCHUNK = B // BC
BN = 2048


def _sc_gather_t(tabT, idx):
    mesh = plsc.VectorSubcoreMesh(core_axis_name="core", subcore_axis_name="subcore")

    @functools.partial(
        pl.kernel,
        out_type=jax.ShapeDtypeStruct((CAT_DIM, B), jnp.float32),
        mesh=mesh,
        compiler_params=pltpu.CompilerParams(needs_layout_passes=False),
        scratch_types=[
            pltpu.VMEM((VOCAB,), jnp.float32),
            pltpu.VMEM((B,), jnp.int32),
            pltpu.VMEM((BC,), jnp.float32),
            pltpu.VMEM((BC,), jnp.float32),
            pltpu.SemaphoreType.DMA,
            pltpu.SemaphoreType.DMA,
            pltpu.SemaphoreType.DMA,
        ],
    )
    def gather_kernel(tabT_hbm, idx_hbm, out_hbm, tab_v, idx_v,
                      out_va, out_vb, sem_i, sem_o0, sem_o1):
        w = lax.axis_index("subcore") * 2 + lax.axis_index("core")
        p = lax.rem(w, 16)
        half = w // 16
        NT_H = NTAB // 2
        t0 = (p * NT_H) // 16
        sem_o = (sem_o0, sem_o1)
        out_bufs = (out_va, out_vb)

        def start_idx(t):
            pltpu.async_copy(idx_hbm.at[t], idx_v, sem_i)

        def wait_idx():
            pltpu.make_async_copy(idx_hbm.at[0], idx_v, sem_i).wait()

        def wait_out(buf):
            pltpu.make_async_copy(
                out_bufs[buf], out_hbm.at[0, pl.ds(0, BC)], sem_o[buf]).wait()

        start_idx(half * NT_H + t0)

        @pl.loop(0, NT_H)
        def _(k):
            t = half * NT_H + lax.rem(t0 + k, NT_H)

            for ei in range(2):
                e = p * 2 + ei
                pltpu.sync_copy(tabT_hbm.at[t, e], tab_v)
                if ei == 0:
                    wait_idx()

                for c in range(NCHUNK):
                    cc = ei * NCHUNK + c
                    buf = cc % 2
                    if cc >= 2:
                        wait_out(buf)
                    else:
                        @pl.when(k > 0)
                        def _():
                            wait_out(buf)

                    ob = out_bufs[buf]

                    @plsc.parallel_loop(0, BC // 16, unroll=16)
                    def _(i):
                        iv = idx_v[pl.ds(c * BC + i * 16, 16)]
                        ob[pl.ds(i * 16, 16)] = plsc.load_gather(tab_v, [iv])

                    pltpu.async_copy(
                        ob, out_hbm.at[t * EDIM + e, pl.ds(c * BC, BC)],
                        sem_o[buf])

            @pl.when(k < NT_H - 1)
            def _():
                start_idx(half * NT_H + lax.rem(t0 + k + 1, NT_H))

        wait_out(0)
        wait_out(1)

    return gather_kernel(tabT, idx)


def _mlp_t_body(xT_ref, numT_ref, w1cT_ref, w1nT_ref, b1_ref, w2T_ref, b2_ref,
                w3T_ref, b3_ref, w4T_ref, b4_ref, outT_ref):
    h = jnp.dot(w1cT_ref[...], xT_ref[...], preferred_element_type=jnp.float32)
    h = h + jnp.dot(w1nT_ref[...], numT_ref[...], preferred_element_type=jnp.float32)
    h = jnp.maximum(h + b1_ref[...], 0.0)
    h = jnp.maximum(
        jnp.dot(w2T_ref[...], h, preferred_element_type=jnp.float32) + b2_ref[...], 0.0)
    h = jnp.maximum(
        jnp.dot(w3T_ref[...], h, preferred_element_type=jnp.float32) + b3_ref[...], 0.0)
    outT_ref[...] = jnp.dot(w4T_ref[...], h, preferred_element_type=jnp.float32) + b4_ref[...]


def _mlp_t(xT, numT, w1cT, w1nT, b1c, w2T, b2c, w3T, b3c, w4T, b4c):
    nblk = B // BN
    full = lambda shape: pl.BlockSpec(shape, lambda i: (0, 0))
    return pl.pallas_call(
        _mlp_t_body,
        grid=(nblk,),
        in_specs=[
            pl.BlockSpec((CAT_DIM, BN), lambda i: (0, i)),
            pl.BlockSpec((NUM_DENSE, BN), lambda i: (0, i)),
            full((512, CAT_DIM)),
            full((512, NUM_DENSE)),
            full((512, 1)),
            full((256, 512)),
            full((256, 1)),
            full((128, 256)),
            full((128, 1)),
            full((1, 128)),
            full((1, 1)),
        ],
        out_specs=pl.BlockSpec((1, BN), lambda i: (0, i)),
        out_shape=jax.ShapeDtypeStruct((1, B), jnp.float32),
    )(xT, numT, w1cT, w1nT, b1c, w2T, b2c, w3T, b3c, w4T, b4c)


def kernel(numerical_features, cat_features, tables, W1, b1, W2, b2, W3, b3, W4, b4):
    idx = jnp.mod(cat_features[:, :, 0], VOCAB)
    tabT = jnp.transpose(tables, (0, 2, 1))
    xT = _sc_gather_t(tabT, idx)
    numT = numerical_features.T
    outT = _mlp_t(
        xT,
        numT,
        W1[:CAT_DIM].T,
        W1[CAT_DIM:].T,
        b1.reshape(-1, 1),
        W2.T,
        b2.reshape(-1, 1),
        W3.T,
        b3.reshape(-1, 1),
        W4.T,
        b4.reshape(-1, 1),
    )
    return outT.reshape(B, 1)
